# baseline (speedup 1.0000x reference)
import functools

import jax
import jax.numpy as jnp
from jax.experimental import pallas as pl
from jax.experimental.pallas import tpu as pltpu

N = 4096
NREL = 5
DIN = 128
DOUT = 128
BLK = 1024


def _gcn_kernel(input_ref, adj_ref, w_ref, att_ref, bias_ref, out_ref,
                support_ref, outputs_ref, colsum_ref):
    i = pl.program_id(0)
    r = pl.program_id(1)
    nblk = pl.num_programs(1)

    @pl.when(r == 0)
    def _compute_support():
        support_ref[:] = jnp.dot(
            input_ref[:], w_ref[0],
            preferred_element_type=jnp.float32).astype(jnp.bfloat16)

    blk = jnp.dot(adj_ref[0].astype(jnp.bfloat16), support_ref[:],
                  preferred_element_type=jnp.float32)
    outputs_ref[pl.ds(i * N + r * BLK, BLK), :] = blk
    csum = jnp.sum(blk, axis=0, keepdims=True)

    @pl.when(r == 0)
    def _init_colsum():
        colsum_ref[pl.ds(i, 1), :] = csum

    @pl.when(r != 0)
    def _acc_colsum():
        colsum_ref[pl.ds(i, 1), :] += csum

    @pl.when((i == NREL - 1) & (r == nblk - 1))
    def _epilogue():
        cs = colsum_ref[:]
        total = jnp.sum(cs, axis=0, keepdims=True)
        rel_norm = cs / total
        logits = att_ref[:] * rel_norm
        m = jnp.max(logits, axis=0, keepdims=True)
        e = jnp.exp(logits - m)
        att = e / jnp.sum(e, axis=0, keepdims=True)
        acc = outputs_ref[0:N, :] * att[0:1, :]
        for k in range(1, NREL):
            acc = acc + outputs_ref[k * N:(k + 1) * N, :] * att[k:k + 1, :]
        out_ref[:] = acc + bias_ref[:]


@functools.partial(jax.jit, static_argnames=("interpret",))
def _gcn(input, adjs, adj_weight, attention, bias, interpret=False):
    att2d = attention.reshape(NREL, 1)
    bias2d = bias.reshape(1, DOUT)
    nblk = N // BLK
    return pl.pallas_call(
        _gcn_kernel,
        grid=(NREL, nblk),
        in_specs=[
            pl.BlockSpec((N, DIN), lambda i, r: (0, 0)),
            pl.BlockSpec((1, BLK, N), lambda i, r: (i, r, 0)),
            pl.BlockSpec((1, DIN, DOUT), lambda i, r: (i, 0, 0)),
            pl.BlockSpec((NREL, 1), lambda i, r: (0, 0)),
            pl.BlockSpec((1, DOUT), lambda i, r: (0, 0)),
        ],
        out_specs=pl.BlockSpec((N, DOUT), lambda i, r: (0, 0)),
        out_shape=jax.ShapeDtypeStruct((N, DOUT), jnp.float32),
        scratch_shapes=[
            pltpu.VMEM((N, DOUT), jnp.bfloat16),
            pltpu.VMEM((NREL * N, DOUT), jnp.float32),
            pltpu.VMEM((NREL, DOUT), jnp.float32),
        ],
        interpret=interpret,
    )(input, adjs, adj_weight, att2d, bias2d)


def kernel(input, adjs, adj_weight, attention, bias):
    return _gcn(input, adjs, adj_weight, attention, bias)


# speedup vs baseline: 1.0327x; 1.0327x over previous
import functools

import jax
import jax.numpy as jnp
from jax.experimental import pallas as pl
from jax.experimental.pallas import tpu as pltpu

N = 4096
NREL = 5
DIN = 128
DOUT = 128
BLK = 512


def _gcn_kernel(input_ref, adj_ref, w_ref, att_ref, bias_ref, out_ref,
                support_ref, outputs_ref, colsum_ref):
    i = pl.program_id(0)
    r = pl.program_id(1)
    nblk = pl.num_programs(1)

    @pl.when(r == 0)
    def _compute_support():
        support_ref[:] = jnp.dot(
            input_ref[:], w_ref[0],
            preferred_element_type=jnp.float32).astype(jnp.bfloat16)

    blk = jax.lax.dot_general(
        adj_ref[0], support_ref[:],
        dimension_numbers=(((1,), (0,)), ((), ())),
        precision=jax.lax.Precision.DEFAULT,
        preferred_element_type=jnp.float32)
    outputs_ref[pl.ds(i * N + r * BLK, BLK), :] = blk.astype(jnp.bfloat16)
    csum = jnp.sum(blk, axis=0, keepdims=True)

    @pl.when(r == 0)
    def _init_colsum():
        colsum_ref[pl.ds(i, 1), :] = csum

    @pl.when(r != 0)
    def _acc_colsum():
        colsum_ref[pl.ds(i, 1), :] += csum

    @pl.when((i == NREL - 1) & (r == nblk - 1))
    def _epilogue():
        cs = colsum_ref[:]
        total = jnp.sum(cs, axis=0, keepdims=True)
        rel_norm = cs / total
        logits = att_ref[:] * rel_norm
        m = jnp.max(logits, axis=0, keepdims=True)
        e = jnp.exp(logits - m)
        att = e / jnp.sum(e, axis=0, keepdims=True)
        acc = outputs_ref[0:N, :] * att[0:1, :]
        for k in range(1, NREL):
            acc = acc + outputs_ref[k * N:(k + 1) * N, :] * att[k:k + 1, :]
        out_ref[:] = acc + bias_ref[:]


@functools.partial(jax.jit, static_argnames=("interpret",))
def _gcn(input, adjs, adj_weight, attention, bias, interpret=False):
    att2d = attention.reshape(NREL, 1)
    bias2d = bias.reshape(1, DOUT)
    nblk = N // BLK
    return pl.pallas_call(
        _gcn_kernel,
        grid=(NREL, nblk),
        in_specs=[
            pl.BlockSpec((N, DIN), lambda i, r: (0, 0)),
            pl.BlockSpec((1, BLK, N), lambda i, r: (i, r, 0)),
            pl.BlockSpec((1, DIN, DOUT), lambda i, r: (i, 0, 0)),
            pl.BlockSpec((NREL, 1), lambda i, r: (0, 0)),
            pl.BlockSpec((1, DOUT), lambda i, r: (0, 0)),
        ],
        out_specs=pl.BlockSpec((N, DOUT), lambda i, r: (0, 0)),
        out_shape=jax.ShapeDtypeStruct((N, DOUT), jnp.float32),
        scratch_shapes=[
            pltpu.VMEM((N, DOUT), jnp.bfloat16),
            pltpu.VMEM((NREL * N, DOUT), jnp.bfloat16),
            pltpu.VMEM((NREL, DOUT), jnp.float32),
        ],
        interpret=interpret,
    )(input, adjs, adj_weight, att2d, bias2d)


def kernel(input, adjs, adj_weight, attention, bias):
    return _gcn(input, adjs, adj_weight, attention, bias)
